# baseline (speedup 1.0000x reference)
import functools

import jax
import jax.numpy as jnp
from jax import lax
from jax.experimental import pallas as pl
from jax.experimental.pallas import tpu as pltpu
from jax.experimental.pallas import tpu_sc as plsc

_NC = 2
_NS = 16
_NW = _NC * _NS


@functools.partial(jax.jit, static_argnums=(2, 3))
def _gather(weight, idx, rows_per_chunk, S):
    R, SP = idx.shape
    D = weight.shape[1]
    rows_per_w = R // _NW
    n_chunks = rows_per_w // rows_per_chunk
    assert n_chunks % 2 == 0 and n_chunks * rows_per_chunk == rows_per_w
    mesh = plsc.VectorSubcoreMesh(core_axis_name="c", subcore_axis_name="s")

    @functools.partial(
        pl.kernel,
        mesh=mesh,
        out_type=jax.ShapeDtypeStruct((R, S, D), jnp.float32),
        scratch_types=[
            pltpu.VMEM((2, rows_per_chunk, SP), jnp.int32),
            pltpu.VMEM((2, rows_per_chunk, SP, D), jnp.float32),
            pltpu.SemaphoreType.DMA((2,)),
            pltpu.SemaphoreType.DMA((2,)),
            pltpu.SemaphoreType.DMA((2,)),
        ],
        compiler_params=pltpu.CompilerParams(use_tc_tiling_on_sc=False),
    )
    def k(table_hbm, idx_hbm, out_hbm, idx_v, rows_v, sem_i, sem_g, sem_s):
        wid = lax.axis_index("s") * _NC + lax.axis_index("c")
        base = wid * rows_per_w

        def idx_copy(c, b):
            return pltpu.make_async_copy(
                idx_hbm.at[pl.ds(base + c * rows_per_chunk, rows_per_chunk)],
                idx_v.at[b], sem_i.at[b])

        def gather_copy(b, r):
            return pltpu.make_async_copy(
                table_hbm.at[idx_v.at[b, r]], rows_v.at[b, r], sem_g.at[b])

        def store_copy(c, b, r):
            return pltpu.make_async_copy(
                rows_v.at[b, r, pl.ds(0, S)],
                out_hbm.at[base + c * rows_per_chunk + r],
                sem_s.at[b])

        idx_copy(0, 0).start()
        idx_copy(1, 1).start()

        def body(i, carry):
            for b in range(2):
                c = 2 * i + b

                @pl.when(c >= 2)
                def _wait_store():
                    for r in range(rows_per_chunk):
                        store_copy(c - 2, b, r).wait()

                idx_copy(c, b).wait()
                for r in range(rows_per_chunk):
                    gather_copy(b, r).start()
                for r in range(rows_per_chunk):
                    gather_copy(b, r).wait()

                @pl.when(c + 2 < n_chunks)
                def _prefetch_idx():
                    idx_copy(c + 2, b).start()

                for r in range(rows_per_chunk):
                    store_copy(c, b, r).start()
            return carry

        lax.fori_loop(0, n_chunks // 2, body, 0)
        for r in range(rows_per_chunk):
            store_copy(n_chunks - 2, 0, r).wait()
            store_copy(n_chunks - 1, 1, r).wait()

    return k(weight, idx)


def kernel(idx, weight):
    S = idx.shape[1]
    idx64 = jnp.pad(idx.astype(jnp.int32), ((0, 0), (0, 64 - S)))
    return _gather(weight, idx64, 8, S)


# speedup vs baseline: 2.0112x; 2.0112x over previous
import functools

import jax
import jax.numpy as jnp
from jax import lax
from jax.experimental import pallas as pl
from jax.experimental.pallas import tpu as pltpu
from jax.experimental.pallas import tpu_sc as plsc

_NC = 2
_NS = 16
_NW = _NC * _NS


@functools.partial(jax.jit, static_argnums=(2,))
def _gather(weight, idx_flat, chunk):
    N = idx_flat.shape[0]
    D = weight.shape[1]
    per_w = N // _NW
    n_chunks = per_w // chunk
    assert n_chunks % 2 == 0 and n_chunks * chunk * _NW == N
    mesh = plsc.VectorSubcoreMesh(core_axis_name="c", subcore_axis_name="s")

    @functools.partial(
        pl.kernel,
        mesh=mesh,
        out_type=jax.ShapeDtypeStruct((N, D), jnp.float32),
        scratch_types=[
            pltpu.VMEM((2, chunk), jnp.int32),
            pltpu.VMEM((2, chunk, D), jnp.float32),
            pltpu.SemaphoreType.DMA((2,)),
            pltpu.SemaphoreType.DMA((2,)),
            pltpu.SemaphoreType.DMA((2,)),
        ],
        compiler_params=pltpu.CompilerParams(use_tc_tiling_on_sc=False),
    )
    def k(table_hbm, idx_hbm, out_hbm, idx_v, rows_v, sem_i, sem_g, sem_s):
        wid = lax.axis_index("s") * _NC + lax.axis_index("c")
        base = wid * per_w

        def idx_copy(c, b):
            return pltpu.make_async_copy(
                idx_hbm.at[pl.ds(base + c * chunk, chunk)],
                idx_v.at[b], sem_i.at[b])

        def gather_copy(b):
            return pltpu.make_async_copy(
                table_hbm.at[idx_v.at[b]], rows_v.at[b], sem_g.at[b])

        def store_copy(c, b):
            return pltpu.make_async_copy(
                rows_v.at[b],
                out_hbm.at[pl.ds(base + c * chunk, chunk)],
                sem_s.at[b])

        idx_copy(0, 0).start()
        idx_copy(1, 1).start()

        def body(i, carry):
            for b in range(2):
                c = 2 * i + b

                @pl.when(c >= 2)
                def _wait_store():
                    store_copy(c - 2, b).wait()

                idx_copy(c, b).wait()
                gather_copy(b).start()
                gather_copy(b).wait()

                @pl.when(c + 2 < n_chunks)
                def _prefetch_idx():
                    idx_copy(c + 2, b).start()

                store_copy(c, b).start()
            return carry

        lax.fori_loop(0, n_chunks // 2, body, 0)
        store_copy(n_chunks - 2, 0).wait()
        store_copy(n_chunks - 1, 1).wait()

    return k(weight, idx_flat)


def kernel(idx, weight):
    R, S = idx.shape
    idx_flat = idx.astype(jnp.int32).reshape(R * S)
    out = _gather(weight, idx_flat, 512)
    return out.reshape(R, S, weight.shape[1])


# speedup vs baseline: 2.0329x; 1.0107x over previous
import functools

import jax
import jax.numpy as jnp
from jax import lax
from jax.experimental import pallas as pl
from jax.experimental.pallas import tpu as pltpu
from jax.experimental.pallas import tpu_sc as plsc

_NC = 2
_NS = 16
_NW = _NC * _NS


@functools.partial(jax.jit, static_argnums=(2,))
def _gather(weight, idx_flat, chunk):
    N = idx_flat.shape[0]
    D = weight.shape[1]
    per_w = N // _NW
    n_chunks = per_w // chunk
    assert n_chunks % 2 == 0 and n_chunks * chunk * _NW == N
    mesh = plsc.VectorSubcoreMesh(core_axis_name="c", subcore_axis_name="s")

    @functools.partial(
        pl.kernel,
        mesh=mesh,
        out_type=jax.ShapeDtypeStruct((N, D), jnp.float32),
        scratch_types=[
            pltpu.VMEM((2, chunk), jnp.int32),
            pltpu.VMEM((2, chunk, D), jnp.float32),
            pltpu.SemaphoreType.DMA((2,)),
            pltpu.SemaphoreType.DMA((2,)),
            pltpu.SemaphoreType.DMA((2,)),
        ],
        compiler_params=pltpu.CompilerParams(use_tc_tiling_on_sc=False),
    )
    def k(table_hbm, idx_hbm, out_hbm, idx_v, rows_v, sem_i, sem_g, sem_s):
        wid = lax.axis_index("s") * _NC + lax.axis_index("c")
        base = wid * per_w

        def idx_copy(c, b):
            return pltpu.make_async_copy(
                idx_hbm.at[pl.ds(base + c * chunk, chunk)],
                idx_v.at[b], sem_i.at[b])

        def gather_copy(b):
            return pltpu.make_async_copy(
                table_hbm.at[idx_v.at[b]], rows_v.at[b], sem_g.at[b])

        def store_copy(c, b):
            return pltpu.make_async_copy(
                rows_v.at[b],
                out_hbm.at[pl.ds(base + c * chunk, chunk)],
                sem_s.at[b])

        idx_copy(0, 0).start()
        idx_copy(1, 1).start()
        idx_copy(0, 0).wait()
        gather_copy(0).start()

        def body(i, carry):
            for b in range(2):
                c = 2 * i + b

                @pl.when(c >= 1)
                def _wait_prev_store():
                    store_copy(c - 1, 1 - b).wait()

                @pl.when(c + 1 < n_chunks)
                def _launch_next_gather():
                    idx_copy(c + 1, 1 - b).wait()
                    gather_copy(1 - b).start()

                gather_copy(b).wait()
                store_copy(c, b).start()

                @pl.when(c + 2 < n_chunks)
                def _prefetch_idx():
                    idx_copy(c + 2, b).start()
            return carry

        lax.fori_loop(0, n_chunks // 2, body, 0)
        store_copy(n_chunks - 1, 1).wait()

    return k(weight, idx_flat)


def kernel(idx, weight):
    R, S = idx.shape
    idx_flat = idx.astype(jnp.int32).reshape(R * S)
    out = _gather(weight, idx_flat, 512)
    return out.reshape(R, S, weight.shape[1])


# speedup vs baseline: 2.0360x; 1.0015x over previous
import functools

import jax
import jax.numpy as jnp
from jax import lax
from jax.experimental import pallas as pl
from jax.experimental.pallas import tpu as pltpu
from jax.experimental.pallas import tpu_sc as plsc

_NC = 2
_NS = 16
_NW = _NC * _NS


@functools.partial(jax.jit, static_argnums=(2,))
def _gather(weight, idx_flat, chunk):
    N = idx_flat.shape[0]
    D = weight.shape[1]
    per_w = N // _NW
    n_chunks = per_w // chunk
    assert n_chunks % 2 == 0 and n_chunks * chunk * _NW == N
    mesh = plsc.VectorSubcoreMesh(core_axis_name="c", subcore_axis_name="s")

    @functools.partial(
        pl.kernel,
        mesh=mesh,
        out_type=jax.ShapeDtypeStruct((N, D), jnp.float32),
        scratch_types=[
            pltpu.VMEM((2, chunk), jnp.int32),
            pltpu.VMEM((2, chunk, D), jnp.float32),
            pltpu.SemaphoreType.DMA((2,)),
            pltpu.SemaphoreType.DMA((2,)),
            pltpu.SemaphoreType.DMA((2,)),
        ],
        compiler_params=pltpu.CompilerParams(use_tc_tiling_on_sc=False),
    )
    def k(table_hbm, idx_hbm, out_hbm, idx_v, rows_v, sem_i, sem_g, sem_s):
        wid = lax.axis_index("s") * _NC + lax.axis_index("c")
        base = wid * per_w

        def idx_copy(c, b):
            return pltpu.make_async_copy(
                idx_hbm.at[pl.ds(base + c * chunk, chunk)],
                idx_v.at[b], sem_i.at[b])

        def gather_copy(b):
            return pltpu.make_async_copy(
                table_hbm.at[idx_v.at[b]], rows_v.at[b], sem_g.at[b])

        def store_copy(c, b):
            return pltpu.make_async_copy(
                rows_v.at[b],
                out_hbm.at[pl.ds(base + c * chunk, chunk)],
                sem_s.at[b])

        idx_copy(0, 0).start()
        idx_copy(1, 1).start()
        idx_copy(0, 0).wait()
        gather_copy(0).start()

        def body(i, carry):
            for b in range(2):
                c = 2 * i + b

                @pl.when(c >= 1)
                def _wait_prev_store():
                    store_copy(c - 1, 1 - b).wait()

                @pl.when(c + 1 < n_chunks)
                def _launch_next_gather():
                    idx_copy(c + 1, 1 - b).wait()
                    gather_copy(1 - b).start()

                gather_copy(b).wait()
                store_copy(c, b).start()

                @pl.when(c + 2 < n_chunks)
                def _prefetch_idx():
                    idx_copy(c + 2, b).start()
            return carry

        lax.fori_loop(0, n_chunks // 2, body, 0)
        store_copy(n_chunks - 1, 1).wait()

    return k(weight, idx_flat)


def kernel(idx, weight):
    R, S = idx.shape
    idx_flat = idx.astype(jnp.int32).reshape(R * S)
    out = _gather(weight, idx_flat, 1600)
    return out.reshape(R, S, weight.shape[1])


# speedup vs baseline: 3.3090x; 1.6252x over previous
import functools

import jax
import jax.numpy as jnp
from jax import lax
from jax.experimental import pallas as pl
from jax.experimental.pallas import tpu as pltpu
from jax.experimental.pallas import tpu_sc as plsc

_NC = 2
_NS = 16
_NW = _NC * _NS


@functools.partial(jax.jit, static_argnums=(2, 3))
def _gather(weight, idx_flat, S, chunk):
    N = idx_flat.shape[0]
    D = weight.shape[1]
    R = N // S
    per_w = R // _NW
    n_chunks = per_w // chunk
    assert n_chunks % 2 == 0 and n_chunks * chunk * _NW == R
    mesh = plsc.VectorSubcoreMesh(core_axis_name="c", subcore_axis_name="s")

    @functools.partial(
        pl.kernel,
        mesh=mesh,
        out_type=jax.ShapeDtypeStruct((R, S, D), jnp.float32),
        scratch_types=[
            pltpu.VMEM((2, chunk * S), jnp.int32),
            pltpu.VMEM((2, chunk * S, D), jnp.float32),
            pltpu.SemaphoreType.DMA((2,)),
            pltpu.SemaphoreType.DMA((2,)),
            pltpu.SemaphoreType.DMA((2,)),
        ],
        compiler_params=pltpu.CompilerParams(use_tc_tiling_on_sc=False),
    )
    def k(table_hbm, idx_hbm, out_hbm, idx_v, rows_v, sem_i, sem_g, sem_s):
        wid = lax.axis_index("s") * _NC + lax.axis_index("c")
        base = wid * per_w

        def idx_copy(c, b):
            return pltpu.make_async_copy(
                idx_hbm.at[pl.ds((base + c * chunk) * S, chunk * S)],
                idx_v.at[b], sem_i.at[b])

        def gather_copy(b):
            return pltpu.make_async_copy(
                table_hbm.at[idx_v.at[b]], rows_v.at[b], sem_g.at[b])

        def store_copy(c, b):
            def one(r):
                return pltpu.make_async_copy(
                    rows_v.at[b, pl.ds(r * S, S)],
                    out_hbm.at[base + c * chunk + r], sem_s.at[b])
            return one

        def store_start(c, b):
            lax.fori_loop(0, chunk, lambda r, _: (store_copy(c, b)(r).start(), 0)[1], 0)

        def store_wait(c, b):
            lax.fori_loop(0, chunk, lambda r, _: (store_copy(c, b)(r).wait(), 0)[1], 0)

        idx_copy(0, 0).start()
        idx_copy(1, 1).start()
        idx_copy(0, 0).wait()
        gather_copy(0).start()

        def body(i, carry):
            for b in range(2):
                c = 2 * i + b

                @pl.when(c >= 1)
                def _wait_prev_store():
                    store_wait(c - 1, 1 - b)

                @pl.when(c + 1 < n_chunks)
                def _launch_next_gather():
                    idx_copy(c + 1, 1 - b).wait()
                    gather_copy(1 - b).start()

                gather_copy(b).wait()
                store_start(c, b)

                @pl.when(c + 2 < n_chunks)
                def _prefetch_idx():
                    idx_copy(c + 2, b).start()
            return carry

        lax.fori_loop(0, n_chunks // 2, body, 0)
        store_wait(n_chunks - 1, 1)

    return k(weight, idx_flat)


def kernel(idx, weight):
    R, S = idx.shape
    idx_flat = idx.astype(jnp.int32).reshape(R * S)
    return _gather(weight, idx_flat, S, 32)
